# baseline (device time: 337005 ns/iter reference)
import jax
import jax.numpy as jnp
from jax import lax
from jax.experimental import pallas as pl
from jax.experimental.pallas import tpu as pltpu

N_DEV = 4
M = 4096
K_SHARD = 1024
N_GLOBAL = 8192
M_CHUNK = M // N_DEV
HALF = M_CHUNK // 2
TILE_N = 1024
STEP_N = 2 * TILE_N
N_STEPS = N_GLOBAL // STEP_N


def kernel(x, w_mat):
    def body(x_ref, w_ref, o_ref, comm_a, comm_b, stage,
             send_a, recv_a, send_b, recv_b, stage_sem):
        d = lax.axis_index("i")
        left = lax.rem(d + N_DEV - 1, N_DEV)
        right = lax.rem(d + 1, N_DEV)

        barrier = pltpu.get_barrier_semaphore()
        pl.semaphore_signal(
            barrier, inc=1, device_id=(left,),
            device_id_type=pl.DeviceIdType.MESH,
        )
        pl.semaphore_signal(
            barrier, inc=1, device_id=(right,),
            device_id_type=pl.DeviceIdType.MESH,
        )
        pl.semaphore_wait(barrier, 2)

        def part(c, col):
            return jnp.dot(
                x_ref[pl.ds(c * M_CHUNK, M_CHUNK), :],
                w_ref[:, pl.ds(col, TILE_N)],
                preferred_element_type=jnp.float32,
            )

        def hop_rdma(comm, sems_s, sems_r, s, h, nbr):
            lo = h * HALF
            return pltpu.make_async_remote_copy(
                src_ref=comm.at[s, lo:lo + HALF],
                dst_ref=comm.at[s + 1, lo:lo + HALF],
                send_sem=sems_s.at[s, h],
                recv_sem=sems_r.at[s, h],
                device_id=(nbr,),
                device_id_type=pl.DeviceIdType.MESH,
            )

        def start_hop(s, h):
            hop_rdma(comm_a, send_a, recv_a, s, h, right).start()
            hop_rdma(comm_b, send_b, recv_b, s, h, left).start()

        def wait_hop(s, h):
            hop_rdma(comm_a, send_a, recv_a, s, h, right).wait()
            hop_rdma(comm_b, send_b, recv_b, s, h, left).wait()

        def add_half(comm, p, s, h):
            lo = h * HALF
            comm[s + 1, lo:lo + HALF, :] = (
                comm[s + 1, lo:lo + HALF, :].astype(jnp.float32)
                + p[lo:lo + HALF, :].astype(jnp.float32)
            ).astype(jnp.bfloat16)

        def silu_half(comm, p, h, col0):
            lo = h * HALF
            y = (
                comm[N_DEV - 1, lo:lo + HALF, :].astype(jnp.float32)
                + p[lo:lo + HALF, :].astype(jnp.float32)
            )
            stage[lo:lo + HALF, col0:col0 + TILE_N] = (
                y * jax.nn.sigmoid(y)
            ).astype(jnp.bfloat16)

        def stage_dma(col):
            return pltpu.make_async_copy(
                stage, o_ref.at[:, pl.ds(col, STEP_N)], stage_sem.at[0],
            )

        ca0 = lax.rem(d + N_DEV - 1, N_DEV)
        cb0 = lax.rem(d + 1, N_DEV)

        comm_a[0, :, :] = part(ca0, 0).astype(jnp.bfloat16)
        comm_b[0, :, :] = part(cb0, TILE_N).astype(jnp.bfloat16)
        start_hop(0, 0)
        start_hop(0, 1)
        pa = part(lax.rem(d + 2 * N_DEV - 2, N_DEV), 0).astype(jnp.bfloat16)
        pb = part(lax.rem(d + 2, N_DEV), TILE_N).astype(jnp.bfloat16)

        def step(st, carry):
            pa, pb = carry
            col_a = st * STEP_N
            col_b = col_a + TILE_N
            col_a_n = lax.rem(col_a + STEP_N, N_GLOBAL)
            col_b_n = col_a_n + TILE_N
            not_last = st < N_STEPS - 1
            for s in range(N_DEV - 2):
                wait_hop(s, 0)
                add_half(comm_a, pa, s, 0)
                add_half(comm_b, pb, s, 0)
                start_hop(s + 1, 0)
                wait_hop(s, 1)
                add_half(comm_a, pa, s, 1)
                add_half(comm_b, pb, s, 1)
                start_hop(s + 1, 1)
                ca = lax.rem(d + 2 * N_DEV - 3 - s, N_DEV)
                cb = lax.rem(d + 3 + s, N_DEV)
                pa = part(ca, col_a).astype(jnp.bfloat16)
                pb = part(cb, col_b).astype(jnp.bfloat16)
                if s == 1:
                    @pl.when(not_last)
                    def _():
                        comm_a[0, :, :] = part(
                            ca0, col_a_n).astype(jnp.bfloat16)
                        comm_b[0, :, :] = part(
                            cb0, col_b_n).astype(jnp.bfloat16)
            wait_hop(N_DEV - 2, 0)

            @pl.when(not_last)
            def _():
                start_hop(0, 0)

            @pl.when(st > 0)
            def _():
                stage_dma(lax.rem(col_a + N_GLOBAL - STEP_N,
                                  N_GLOBAL)).wait()

            silu_half(comm_a, pa, 0, 0)
            silu_half(comm_b, pb, 0, TILE_N)
            wait_hop(N_DEV - 2, 1)

            @pl.when(not_last)
            def _():
                start_hop(0, 1)

            silu_half(comm_a, pa, 1, 0)
            silu_half(comm_b, pb, 1, TILE_N)
            stage_dma(col_a).start()
            pa = part(lax.rem(d + 2 * N_DEV - 2, N_DEV),
                      col_a_n).astype(jnp.bfloat16)
            pb = part(lax.rem(d + 2, N_DEV), col_b_n).astype(jnp.bfloat16)
            return pa, pb

        lax.fori_loop(0, N_STEPS, step, (pa, pb), unroll=False)
        stage_dma((N_STEPS - 1) * STEP_N).wait()

    return pl.pallas_call(
        body,
        in_specs=[
            pl.BlockSpec(memory_space=pltpu.MemorySpace.VMEM),
            pl.BlockSpec(memory_space=pltpu.MemorySpace.VMEM),
        ],
        out_specs=pl.BlockSpec(memory_space=pl.ANY),
        out_shape=jax.ShapeDtypeStruct((M_CHUNK, N_GLOBAL), jnp.bfloat16),
        scratch_shapes=[
            pltpu.VMEM((N_DEV, M_CHUNK, TILE_N), jnp.bfloat16),
            pltpu.VMEM((N_DEV, M_CHUNK, TILE_N), jnp.bfloat16),
            pltpu.VMEM((M_CHUNK, STEP_N), jnp.bfloat16),
            pltpu.SemaphoreType.DMA((N_DEV - 1, 2)),
            pltpu.SemaphoreType.DMA((N_DEV - 1, 2)),
            pltpu.SemaphoreType.DMA((N_DEV - 1, 2)),
            pltpu.SemaphoreType.DMA((N_DEV - 1, 2)),
            pltpu.SemaphoreType.DMA((1,)),
        ],
        compiler_params=pltpu.CompilerParams(
            collective_id=0,
            vmem_limit_bytes=62 * 1024 * 1024,
        ),
    )(x.astype(jnp.bfloat16), w_mat.astype(jnp.bfloat16))
